# baseline (speedup 1.0000x reference)
import functools

import jax
import jax.numpy as jnp
from jax import lax
from jax.experimental import pallas as pl
from jax.experimental.pallas import tpu as pltpu
from jax.experimental.pallas import tpu_sc as plsc

_NC = 2
_NS = 16
_NW = _NC * _NS
_CH = 128
_NBUF = 8


def _sc_embedding_gather(table, ids3):
    nw, n_chunks, ch = ids3.shape
    d = table.shape[1]
    b = nw * n_chunks * ch
    rows_per_w = n_chunks * ch
    n_rounds = n_chunks // _NBUF
    assert n_chunks % _NBUF == 0
    mesh = plsc.VectorSubcoreMesh(core_axis_name="c", subcore_axis_name="s")

    @functools.partial(
        pl.kernel,
        mesh=mesh,
        out_type=jax.ShapeDtypeStruct((b, d), jnp.bfloat16),
        scratch_types=(
            [pltpu.VMEM((n_chunks, ch), jnp.int32)]
            + [pltpu.VMEM((ch, d), jnp.bfloat16) for _ in range(_NBUF)]
            + [pltpu.SemaphoreType.DMA for _ in range(2 * _NBUF)]
        ),
        compiler_params=pltpu.CompilerParams(use_tc_tiling_on_sc=False),
    )
    def k(table_hbm, idx_hbm, out_hbm, idx_v, *scratch):
        bufs = scratch[:_NBUF]
        sem_g = scratch[_NBUF:2 * _NBUF]
        sem_w = scratch[2 * _NBUF:]
        wid = lax.axis_index("s") * _NC + lax.axis_index("c")
        base = wid * rows_per_w
        pltpu.sync_copy(idx_hbm.at[wid], idx_v)

        def fire_gather(slot, c):
            pltpu.async_copy(table_hbm.at[idx_v.at[c]], bufs[slot], sem_g[slot])

        for slot in range(_NBUF):
            fire_gather(slot, slot)

        def round_body(g, carry):
            cbase = g * _NBUF
            for slot in range(_NBUF):
                pltpu.make_async_copy(
                    table_hbm.at[idx_v.at[cbase + slot]], bufs[slot], sem_g[slot]
                ).wait()
                pltpu.async_copy(
                    bufs[slot],
                    out_hbm.at[pl.ds(base + (cbase + slot) * ch, ch)],
                    sem_w[slot],
                )
            for slot in range(_NBUF):
                pltpu.make_async_copy(
                    bufs[slot],
                    out_hbm.at[pl.ds(base + (cbase + slot) * ch, ch)],
                    sem_w[slot],
                ).wait()

                @pl.when(g < n_rounds - 1)
                def _():
                    fire_gather(slot, cbase + _NBUF + slot)

            return carry

        lax.fori_loop(0, n_rounds, round_body, 0)

    return k(table, ids3)


def kernel(token_ids, embedding_table):
    batch, hist = token_ids.shape
    d = embedding_table.shape[1]
    ids = token_ids.reshape(_NW, -1, _CH).astype(jnp.int32)
    out = _sc_embedding_gather(embedding_table.astype(jnp.bfloat16), ids)
    return out.astype(jnp.float32).reshape(batch, hist, d)


# speedup vs baseline: 1.5576x; 1.5576x over previous
import functools

import jax
import jax.numpy as jnp
from jax import lax
from jax.experimental import pallas as pl
from jax.experimental.pallas import tpu as pltpu
from jax.experimental.pallas import tpu_sc as plsc

_NC = 2
_NS = 16
_NW = _NC * _NS
_CH = 128
_NBUF = 8


def _sc_embedding_gather(table, ids3):
    nw, n_chunks, ch = ids3.shape
    d = table.shape[1]
    b = nw * n_chunks * ch
    rows_per_w = n_chunks * ch
    n_rounds = n_chunks // _NBUF
    assert n_chunks % _NBUF == 0
    mesh = plsc.VectorSubcoreMesh(core_axis_name="c", subcore_axis_name="s")

    @functools.partial(
        pl.kernel,
        mesh=mesh,
        out_type=jax.ShapeDtypeStruct((b, d), jnp.float32),
        scratch_types=(
            [pltpu.VMEM((n_chunks, ch), jnp.int32)]
            + [pltpu.VMEM((ch, d), jnp.float32) for _ in range(_NBUF)]
            + [pltpu.SemaphoreType.DMA for _ in range(2 * _NBUF)]
        ),
        compiler_params=pltpu.CompilerParams(use_tc_tiling_on_sc=False),
    )
    def k(table_hbm, idx_hbm, out_hbm, idx_v, *scratch):
        bufs = scratch[:_NBUF]
        sem_g = scratch[_NBUF:2 * _NBUF]
        sem_w = scratch[2 * _NBUF:]
        wid = lax.axis_index("s") * _NC + lax.axis_index("c")
        base = wid * rows_per_w
        pltpu.sync_copy(idx_hbm.at[wid], idx_v)

        def fire_gather(slot, c):
            pltpu.async_copy(table_hbm.at[idx_v.at[c]], bufs[slot], sem_g[slot])

        for slot in range(_NBUF):
            fire_gather(slot, slot)

        def round_body(g, carry):
            cbase = g * _NBUF
            for slot in range(_NBUF):
                pltpu.make_async_copy(
                    table_hbm.at[idx_v.at[cbase + slot]], bufs[slot], sem_g[slot]
                ).wait()
                pltpu.async_copy(
                    bufs[slot],
                    out_hbm.at[pl.ds(base + (cbase + slot) * ch, ch)],
                    sem_w[slot],
                )
            for slot in range(_NBUF):
                pltpu.make_async_copy(
                    bufs[slot],
                    out_hbm.at[pl.ds(base + (cbase + slot) * ch, ch)],
                    sem_w[slot],
                ).wait()

                @pl.when(g < n_rounds - 1)
                def _():
                    fire_gather(slot, cbase + _NBUF + slot)

            return carry

        lax.fori_loop(0, n_rounds, round_body, 0)

    return k(table, ids3)


def kernel(token_ids, embedding_table):
    batch, hist = token_ids.shape
    d = embedding_table.shape[1]
    ids = token_ids.reshape(_NW, -1, _CH).astype(jnp.int32)
    out = _sc_embedding_gather(embedding_table, ids)
    return out.reshape(batch, hist, d)


# speedup vs baseline: 1.5600x; 1.0016x over previous
import functools

import jax
import jax.numpy as jnp
from jax import lax
from jax.experimental import pallas as pl
from jax.experimental.pallas import tpu as pltpu
from jax.experimental.pallas import tpu_sc as plsc

_NC = 2
_NS = 16
_NW = _NC * _NS
_CH = 128
_NBUF = 10


def _sc_embedding_gather(table, ids3):
    nw, n_chunks, ch = ids3.shape
    d = table.shape[1]
    b = nw * n_chunks * ch
    rows_per_w = n_chunks * ch
    n_rounds = n_chunks // _NBUF
    assert n_chunks % _NBUF == 0
    mesh = plsc.VectorSubcoreMesh(core_axis_name="c", subcore_axis_name="s")

    @functools.partial(
        pl.kernel,
        mesh=mesh,
        out_type=jax.ShapeDtypeStruct((b, d), jnp.float32),
        scratch_types=(
            [pltpu.VMEM((n_chunks, ch), jnp.int32)]
            + [pltpu.VMEM((ch, d), jnp.float32) for _ in range(_NBUF)]
            + [pltpu.SemaphoreType.DMA for _ in range(2 * _NBUF)]
        ),
        compiler_params=pltpu.CompilerParams(use_tc_tiling_on_sc=False),
    )
    def k(table_hbm, idx_hbm, out_hbm, idx_v, *scratch):
        bufs = scratch[:_NBUF]
        sem_g = scratch[_NBUF:2 * _NBUF]
        sem_w = scratch[2 * _NBUF:]
        wid = lax.axis_index("s") * _NC + lax.axis_index("c")
        base = wid * rows_per_w
        pltpu.sync_copy(idx_hbm.at[wid], idx_v)

        def fire_gather(slot, c):
            pltpu.async_copy(table_hbm.at[idx_v.at[c]], bufs[slot], sem_g[slot])

        for slot in range(_NBUF):
            fire_gather(slot, slot)

        def round_body(g, carry):
            cbase = g * _NBUF
            for slot in range(_NBUF):
                pltpu.make_async_copy(
                    table_hbm.at[idx_v.at[cbase + slot]], bufs[slot], sem_g[slot]
                ).wait()
                pltpu.async_copy(
                    bufs[slot],
                    out_hbm.at[pl.ds(base + (cbase + slot) * ch, ch)],
                    sem_w[slot],
                )
            for slot in range(_NBUF):
                pltpu.make_async_copy(
                    bufs[slot],
                    out_hbm.at[pl.ds(base + (cbase + slot) * ch, ch)],
                    sem_w[slot],
                ).wait()

                @pl.when(g < n_rounds - 1)
                def _():
                    fire_gather(slot, cbase + _NBUF + slot)

            return carry

        lax.fori_loop(0, n_rounds, round_body, 0)

    return k(table, ids3)


def kernel(token_ids, embedding_table):
    batch, hist = token_ids.shape
    d = embedding_table.shape[1]
    ids = token_ids.reshape(_NW, -1, _CH).astype(jnp.int32)
    out = _sc_embedding_gather(embedding_table, ids)
    return out.reshape(batch, hist, d)
